# Initial kernel scaffold; baseline (speedup 1.0000x reference)
import jax
import jax.numpy as jnp
from jax.experimental import pallas as pl


def kernel(x, width_table, batch_max_seq_len):
    raise NotImplementedError("write your pallas kernel here")


# speedup vs baseline: 1.0411x; 1.0411x over previous
import functools

import jax
import jax.numpy as jnp
import numpy as np
from jax import lax
from jax.experimental import pallas as pl
from jax.experimental.pallas import tpu as pltpu
from jax.experimental.pallas import tpu_sc as plsc

_L = 512
_D = 768
_WD = 64
_F = 2 * _D + _WD
_B = 8
_NW = 8
_K = 64
_NCH = 8

_BUCKET_BINS = [0, 1, 2, 3, 4, 5, 7, 8, 15, 16, 31, 32, 63, 64]


def _span_index_consts():
    starts, ends = [], []
    for w in range(1, _NW + 1):
        for i in range(0, _L - w + 1):
            starts.append(i)
            ends.append(i + w - 1)
    return (np.array(starts, dtype=np.int32), np.array(ends, dtype=np.int32))


_STARTS_NP, _ENDS_NP = _span_index_consts()
_NS = _STARTS_NP.shape[0]


def _sc_body(x_hbm, wt_hbm, out_hbm, wrow_v, buf_v, sem):
    wid = lax.axis_index("s") * 2 + lax.axis_index("c")
    for j in range(2):
        task = wid * 2 + j
        b = task // _NW
        w = task % _NW + 1
        cnt = 513 - w
        off = 513 * (w - 1) - (w - 1) * w // 2
        bucket = w - (w > 5).astype(jnp.int32)

        pltpu.sync_copy(wt_hbm.at[bucket], wrow_v)

        def _fill(r, carry):
            for k2 in range(_WD // 16):
                buf_v[r, pl.ds(2 * _D + 16 * k2, 16)] = wrow_v[pl.ds(16 * k2, 16)]
            return carry

        lax.fori_loop(0, _K, _fill, 0)

        def _chunk(i, carry):
            i0 = jnp.minimum(i * _K, cnt - _K)
            c1 = pltpu.async_copy(
                x_hbm.at[b, pl.ds(i0, _K)], buf_v.at[:, pl.ds(0, _D)], sem)
            c2 = pltpu.async_copy(
                x_hbm.at[b, pl.ds(i0 + w - 1, _K)], buf_v.at[:, pl.ds(_D, _D)], sem)
            c1.wait()
            c2.wait()
            pltpu.sync_copy(buf_v, out_hbm.at[b, pl.ds(off + i0, _K)])
            return carry

        lax.fori_loop(0, _NCH, _chunk, 0)


@functools.partial(
    pl.kernel,
    mesh=plsc.VectorSubcoreMesh(core_axis_name="c", subcore_axis_name="s"),
    out_type=jax.ShapeDtypeStruct((_B, _NS, _F), jnp.float32),
    scratch_types=[
        pltpu.VMEM((_WD,), jnp.float32),
        pltpu.VMEM((_K, _F), jnp.float32),
        pltpu.SemaphoreType.DMA,
    ],
    compiler_params=pltpu.CompilerParams(use_tc_tiling_on_sc=False),
)
def _span_sc(x_hbm, wt_hbm, out_hbm, wrow_v, buf_v, sem):
    _sc_body(x_hbm, wt_hbm, out_hbm, wrow_v, buf_v, sem)


def kernel(x, width_table, batch_max_seq_len):
    spans = _span_sc(x, width_table)
    residual = jnp.asarray(batch_max_seq_len, jnp.int32) - jnp.int32(_L)
    span_indices = jnp.stack(
        [jnp.asarray(_STARTS_NP) + residual, jnp.asarray(_ENDS_NP)], axis=1)
    return (spans, span_indices)


# speedup vs baseline: 11.8059x; 11.3394x over previous
import jax
import jax.numpy as jnp
import numpy as np
from jax.experimental import pallas as pl

_L = 512
_D = 768
_WD = 64
_F = 2 * _D + _WD
_B = 8
_NW = 8
_ST = 512
_NT = 8

_BUCKET_BINS = [0, 1, 2, 3, 4, 5, 7, 8, 15, 16, 31, 32, 63, 64]


def _bucket_of(width):
    return max(ix for ix, v in enumerate(_BUCKET_BINS) if width >= v)


def _span_index_consts():
    starts, ends = [], []
    for w in range(1, _NW + 1):
        for i in range(0, _L - w + 1):
            starts.append(i)
            ends.append(i + w - 1)
    return (np.array(starts, dtype=np.int32), np.array(ends, dtype=np.int32))


_STARTS_NP, _ENDS_NP = _span_index_consts()
_NS = _STARTS_NP.shape[0]
_OFFS = [0]
for _w in range(1, _NW + 1):
    _OFFS.append(_OFFS[-1] + (_L - _w + 1))


def _tile_segments(t):
    lo, hi = _ST * t, min(_ST * (t + 1), _NS)
    segs = []
    for w in range(1, _NW + 1):
        s_lo, s_hi = max(lo, _OFFS[w - 1]), min(hi, _OFFS[w])
        if s_lo < s_hi:
            segs.append((w, s_lo, s_hi))
    return segs


def _tc_body(xt_ref, wt_ref, out_ref):
    t = pl.program_id(1)
    for tt in range(_NT):
        @pl.when(t == tt)
        def _(tt=tt):
            for w, s_lo, s_hi in _tile_segments(tt):
                a = s_lo - _ST * tt
                c = s_hi - s_lo
                s0 = s_lo - _OFFS[w - 1]
                out_ref[0, 0:_D, a:a + c] = xt_ref[0, :, s0:s0 + c]
                out_ref[0, _D:2 * _D, a:a + c] = (
                    xt_ref[0, :, s0 + w - 1:s0 + w - 1 + c])
                wrow = wt_ref[_bucket_of(w), :]
                out_ref[0, 2 * _D:_F, a:a + c] = jnp.broadcast_to(
                    wrow[:, None], (_WD, c))


def kernel(x, width_table, batch_max_seq_len):
    xt = jnp.swapaxes(x, 1, 2)
    spans_t = pl.pallas_call(
        _tc_body,
        grid=(_B, _NT),
        in_specs=[
            pl.BlockSpec((1, _D, _L), lambda b, t: (b, 0, 0)),
            pl.BlockSpec((14, _WD), lambda b, t: (0, 0)),
        ],
        out_specs=pl.BlockSpec((1, _F, _ST), lambda b, t: (b, 0, t)),
        out_shape=jax.ShapeDtypeStruct((_B, _F, _NS), jnp.float32),
    )(xt, width_table)
    spans = jnp.swapaxes(spans_t, 1, 2)
    residual = jnp.asarray(batch_max_seq_len, jnp.int32) - jnp.int32(_L)
    span_indices = jnp.stack(
        [jnp.asarray(_STARTS_NP) + residual, jnp.asarray(_ENDS_NP)], axis=1)
    return (spans, span_indices)


# speedup vs baseline: 13.8596x; 1.1740x over previous
import jax
import jax.numpy as jnp
import numpy as np
from jax.experimental import pallas as pl

_L = 512
_D = 768
_WD = 64
_F = 2 * _D + _WD
_B = 8
_NW = 8
_ST = 1024
_NT = 4

_BUCKET_BINS = [0, 1, 2, 3, 4, 5, 7, 8, 15, 16, 31, 32, 63, 64]


def _bucket_of(width):
    return max(ix for ix, v in enumerate(_BUCKET_BINS) if width >= v)


def _span_index_consts():
    starts, ends = [], []
    for w in range(1, _NW + 1):
        for i in range(0, _L - w + 1):
            starts.append(i)
            ends.append(i + w - 1)
    return (np.array(starts, dtype=np.int32), np.array(ends, dtype=np.int32))


_STARTS_NP, _ENDS_NP = _span_index_consts()
_NS = _STARTS_NP.shape[0]
_OFFS = [0]
for _w in range(1, _NW + 1):
    _OFFS.append(_OFFS[-1] + (_L - _w + 1))


def _tile_segments(t):
    lo, hi = _ST * t, min(_ST * (t + 1), _NS)
    segs = []
    for w in range(1, _NW + 1):
        s_lo, s_hi = max(lo, _OFFS[w - 1]), min(hi, _OFFS[w])
        if s_lo < s_hi:
            segs.append((w, s_lo, s_hi))
    return segs


def _tc_body(xt_ref, wt_ref, out_ref):
    t = pl.program_id(1)
    for tt in range(_NT):
        @pl.when(t == tt)
        def _(tt=tt):
            for w, s_lo, s_hi in _tile_segments(tt):
                a = s_lo - _ST * tt
                c = s_hi - s_lo
                s0 = s_lo - _OFFS[w - 1]
                out_ref[0, 0:_D, a:a + c] = xt_ref[0, :, s0:s0 + c]
                out_ref[0, _D:2 * _D, a:a + c] = (
                    xt_ref[0, :, s0 + w - 1:s0 + w - 1 + c])
                wrow = wt_ref[_bucket_of(w), :]
                out_ref[0, 2 * _D:_F, a:a + c] = jnp.broadcast_to(
                    wrow[:, None], (_WD, c))


def kernel(x, width_table, batch_max_seq_len):
    xt = jnp.swapaxes(x, 1, 2)
    spans_t = pl.pallas_call(
        _tc_body,
        grid=(_B, _NT),
        in_specs=[
            pl.BlockSpec((1, _D, _L), lambda b, t: (b, 0, 0)),
            pl.BlockSpec((14, _WD), lambda b, t: (0, 0)),
        ],
        out_specs=pl.BlockSpec((1, _F, _ST), lambda b, t: (b, 0, t)),
        out_shape=jax.ShapeDtypeStruct((_B, _F, _NS), jnp.float32),
    )(xt, width_table)
    spans = jnp.swapaxes(spans_t, 1, 2)
    residual = jnp.asarray(batch_max_seq_len, jnp.int32) - jnp.int32(_L)
    span_indices = jnp.stack(
        [jnp.asarray(_STARTS_NP) + residual, jnp.asarray(_ENDS_NP)], axis=1)
    return (spans, span_indices)


# speedup vs baseline: 14.6251x; 1.0552x over previous
import jax
import jax.numpy as jnp
import numpy as np
from jax.experimental import pallas as pl

_L = 512
_D = 768
_WD = 64
_F = 2 * _D + _WD
_B = 8
_NW = 8
_ST = 2048
_NT = 2

_BUCKET_BINS = [0, 1, 2, 3, 4, 5, 7, 8, 15, 16, 31, 32, 63, 64]


def _bucket_of(width):
    return max(ix for ix, v in enumerate(_BUCKET_BINS) if width >= v)


def _span_index_consts():
    starts, ends = [], []
    for w in range(1, _NW + 1):
        for i in range(0, _L - w + 1):
            starts.append(i)
            ends.append(i + w - 1)
    return (np.array(starts, dtype=np.int32), np.array(ends, dtype=np.int32))


_STARTS_NP, _ENDS_NP = _span_index_consts()
_NS = _STARTS_NP.shape[0]
_OFFS = [0]
for _w in range(1, _NW + 1):
    _OFFS.append(_OFFS[-1] + (_L - _w + 1))


def _tile_segments(t):
    lo, hi = _ST * t, min(_ST * (t + 1), _NS)
    segs = []
    for w in range(1, _NW + 1):
        s_lo, s_hi = max(lo, _OFFS[w - 1]), min(hi, _OFFS[w])
        if s_lo < s_hi:
            segs.append((w, s_lo, s_hi))
    return segs


def _tc_body(xt_ref, wt_ref, out_ref):
    t = pl.program_id(1)
    for tt in range(_NT):
        @pl.when(t == tt)
        def _(tt=tt):
            for w, s_lo, s_hi in _tile_segments(tt):
                a = s_lo - _ST * tt
                c = s_hi - s_lo
                s0 = s_lo - _OFFS[w - 1]
                out_ref[0, 0:_D, a:a + c] = xt_ref[0, :, s0:s0 + c]
                out_ref[0, _D:2 * _D, a:a + c] = (
                    xt_ref[0, :, s0 + w - 1:s0 + w - 1 + c])
                wrow = wt_ref[_bucket_of(w), :]
                out_ref[0, 2 * _D:_F, a:a + c] = jnp.broadcast_to(
                    wrow[:, None], (_WD, c))


def kernel(x, width_table, batch_max_seq_len):
    xt = jnp.swapaxes(x, 1, 2)
    spans_t = pl.pallas_call(
        _tc_body,
        grid=(_B, _NT),
        in_specs=[
            pl.BlockSpec((1, _D, _L), lambda b, t: (b, 0, 0)),
            pl.BlockSpec((14, _WD), lambda b, t: (0, 0)),
        ],
        out_specs=pl.BlockSpec((1, _F, _ST), lambda b, t: (b, 0, t)),
        out_shape=jax.ShapeDtypeStruct((_B, _F, _NS), jnp.float32),
    )(xt, width_table)
    spans = jnp.swapaxes(spans_t, 1, 2)
    residual = jnp.asarray(batch_max_seq_len, jnp.int32) - jnp.int32(_L)
    span_indices = jnp.stack(
        [jnp.asarray(_STARTS_NP) + residual, jnp.asarray(_ENDS_NP)], axis=1)
    return (spans, span_indices)
